# baseline (speedup 1.0000x reference)
import jax
import jax.numpy as jnp
from jax import lax
from jax.experimental import pallas as pl
from jax.experimental.pallas import tpu as pltpu
from jax.experimental.pallas import tpu_sc as plsc

R = 128
N = 100000
L = 16
NC = 2
NS = 16
NW = NC * NS
NB = R // L
SUB = 8

TC_BLK = 512
TC_COLS = 53760
TC_STEPS = TC_COLS // TC_BLK

SC_TILE0 = TC_COLS // SUB
SC_TILES = N // SUB - SC_TILE0
W_TILES = SC_TILES // NW
REM = SC_TILES - W_TILES * NW
CHUNK_TILES = 30
NCHUNKS = W_TILES // CHUNK_TILES
CHUNK_ROWS = CHUNK_TILES * SUB
NEG_TINY_BITS = -2147483647


def _sc_pre(pt, tgt, thrf, gath, tgtv, thrv, semg):
    wid = lax.axis_index("s") * NC + lax.axis_index("c")

    @pl.when(wid == 0)
    def _():
        lanes = lax.iota(jnp.int32, L)
        pltpu.sync_copy(tgt, tgtv)
        pltpu.async_copy(pt.at[tgtv], gath, semg).wait()
        for m in range(NB):
            v = jnp.zeros((L,), jnp.float32)
            for l in range(L):
                row = gath[m * L + l, pl.ds(m * L, L)]
                v = jnp.where(lanes == jnp.int32(l), row, v)
            b = lax.bitcast_convert_type(v, jnp.int32)
            blo = jnp.where(v > jnp.float32(0.0), b - 1, b + 1)
            blo = jnp.where(v == jnp.float32(0.0), jnp.int32(NEG_TINY_BITS),
                            blo)
            thrv[0, pl.ds(m * L, L)] = lax.bitcast_convert_type(
                blo, jnp.float32)
            thrv[1, pl.ds(m * L, L)] = v
        for j in range(2, SUB):
            thrv[j, pl.ds(0, L)] = jnp.zeros((L,), jnp.float32)
        pltpu.sync_copy(thrv, thrf)


def _sc_stage1(pt, tgt, thrf, outp, buf0, buf1, tgtv, thrv, ebuf, outv,
               sem0, sem1, semg):
    wid = lax.axis_index("s") * NC + lax.axis_index("c")
    lanes = lax.iota(jnp.int32, L) + wid * jnp.int32(0)

    pltpu.sync_copy(tgt, tgtv)
    pltpu.sync_copy(thrf, thrv)

    t_vec, thr_lo, thr_hi, acc = [], [], [], []
    for m in range(NB):
        thr_lo.append(thrv[0, pl.ds(m * L, L)])
        thr_hi.append(thrv[1, pl.ds(m * L, L)])
        t_vec.append(tgtv[pl.ds(m * L, L)])
        acc.append(lanes * jnp.int32(0))

    def chunk_src(c):
        row0 = (SC_TILE0 + wid * W_TILES + c * CHUNK_TILES) * SUB
        return pt.at[pl.ds(row0, CHUNK_ROWS), pl.ds(0, R)]

    bufs = (buf0, buf1)
    sems = (sem0, sem1)
    copies = {0: pltpu.async_copy(chunk_src(0), bufs[0], sems[0])}

    def count_block(buf, nrows, col0, accs, lo, hi):
        cs0 = jnp.broadcast_to(col0, (L,))

        @plsc.parallel_loop(0, nrows, 1, carry=tuple(accs) + (cs0,))
        def _loop(s, carry):
            a = list(carry[:NB])
            cs = carry[NB]
            for m in range(NB):
                x = buf[s, pl.ds(m * L, L)]
                thr = jnp.where(cs < t_vec[m], lo[m], hi[m])
                a[m] = a[m] + jnp.where(x > thr, jnp.int32(1), jnp.int32(0))
            return tuple(a) + (cs + jnp.int32(1),)

        return list(_loop[:NB])

    for c in range(NCHUNKS):
        p = c % 2
        if c + 1 < NCHUNKS:
            copies[c + 1] = pltpu.async_copy(chunk_src(c + 1),
                                             bufs[1 - p], sems[1 - p])
        copies[c].wait()
        acc = count_block(bufs[p], CHUNK_ROWS,
                          (SC_TILE0 + wid * W_TILES + c * CHUNK_TILES) * SUB,
                          acc, thr_lo, thr_hi)

    en = jnp.broadcast_to(wid < REM, (L,))
    row0 = (SC_TILE0 + NW * W_TILES + jnp.minimum(wid, REM - 1)) * SUB
    pltpu.async_copy(pt.at[pl.ds(row0, SUB), pl.ds(0, R)],
                     ebuf, semg).wait()
    inf = jnp.float32(float("inf"))
    lo_e = [jnp.where(en, thr_lo[m], inf) for m in range(NB)]
    hi_e = [jnp.where(en, thr_hi[m], inf) for m in range(NB)]
    acc = count_block(ebuf, SUB, row0, acc, lo_e, hi_e)

    for m in range(NB):
        outv[0, pl.ds(m * L, L)] = acc[m]
    pltpu.sync_copy(outv, outp.at[pl.ds(0, SUB),
                                  pl.ds(pl.multiple_of(wid * R, R), R)])


def _tc_count(thrf_ref, meta_ref, x_ref, o_ref):
    i = pl.program_id(0)

    @pl.when(i == 0)
    def _():
        o_ref[...] = jnp.zeros((SUB, R), jnp.int32)

    x = x_ref[...]
    cols = i * TC_BLK + lax.broadcasted_iota(jnp.int32, (TC_BLK, R), 0)
    thr = jnp.where(cols < meta_ref[0:1, :], thrf_ref[0:1, :],
                    thrf_ref[1:2, :])
    cnt = (x > thr).astype(jnp.int32).reshape(TC_BLK // SUB, SUB, R)
    o_ref[...] += jnp.sum(cnt, axis=0)


def _sc_stage2(outp, tcp, meta, out, pv, tcv, metav, outv, semc):
    wid = lax.axis_index("s") * NC + lax.axis_index("c")

    @pl.when(wid == 0)
    def _():
        pltpu.sync_copy(outp, pv)
        pltpu.sync_copy(tcp, tcv)
        pltpu.sync_copy(meta, metav)
        kthr = jnp.broadcast_to(
            lax.reduce_max(metav[1, pl.ds(0, L)], axes=(0,)), (L,))
        correct = jnp.zeros((L,), jnp.int32)
        for m in range(NB):
            rank = pv[0, pl.ds(m * L, L)]
            for w in range(1, NW):
                rank = rank + pv[0, pl.ds(w * R + m * L, L)]
            for j in range(SUB):
                rank = rank + tcv[j, pl.ds(m * L, L)]
            correct = correct + jnp.where(rank < kthr, jnp.int32(1),
                                          jnp.int32(0))
        total = lax.reduce_sum(correct, axes=(0,))
        outv[0, pl.ds(0, L)] = jnp.broadcast_to(
            total.astype(jnp.float32), (L,))
        for j in range(1, SUB):
            outv[j, pl.ds(0, L)] = jnp.zeros((L,), jnp.float32)
        pltpu.sync_copy(outv, out)


@jax.jit
def _run(pt, meta, tgt):
    mesh = plsc.VectorSubcoreMesh(core_axis_name="c", subcore_axis_name="s")
    params = pltpu.CompilerParams(needs_layout_passes=False,
                                  use_tc_tiling_on_sc=True)
    pre = pl.kernel(
        _sc_pre,
        out_type=jax.ShapeDtypeStruct((SUB, R), jnp.float32),
        mesh=mesh,
        compiler_params=params,
        scratch_types=[
            pltpu.VMEM((R, R), jnp.float32),
            pltpu.VMEM((R,), jnp.int32),
            pltpu.VMEM((SUB, R), jnp.float32),
            pltpu.SemaphoreType.DMA,
        ],
    )
    thrf = pre(pt, tgt)

    s1 = pl.kernel(
        _sc_stage1,
        out_type=jax.ShapeDtypeStruct((SUB, NW * R), jnp.int32),
        mesh=mesh,
        compiler_params=params,
        scratch_types=[
            pltpu.VMEM((CHUNK_ROWS, R), jnp.float32),
            pltpu.VMEM((CHUNK_ROWS, R), jnp.float32),
            pltpu.VMEM((R,), jnp.int32),
            pltpu.VMEM((SUB, R), jnp.float32),
            pltpu.VMEM((SUB, R), jnp.float32),
            pltpu.VMEM((SUB, R), jnp.int32),
            pltpu.SemaphoreType.DMA,
            pltpu.SemaphoreType.DMA,
            pltpu.SemaphoreType.DMA,
        ],
    )
    outp = s1(pt, tgt, thrf)

    tcp = pl.pallas_call(
        _tc_count,
        grid=(TC_STEPS,),
        in_specs=[
            pl.BlockSpec((SUB, R), lambda i: (0, 0)),
            pl.BlockSpec((SUB, R), lambda i: (0, 0)),
            pl.BlockSpec((TC_BLK, R), lambda i: (i, 0)),
        ],
        out_specs=pl.BlockSpec((SUB, R), lambda i: (0, 0)),
        out_shape=jax.ShapeDtypeStruct((SUB, R), jnp.int32),
    )(thrf, meta, pt)

    s2 = pl.kernel(
        _sc_stage2,
        out_type=jax.ShapeDtypeStruct((SUB, R), jnp.float32),
        mesh=mesh,
        compiler_params=params,
        scratch_types=[
            pltpu.VMEM((SUB, NW * R), jnp.int32),
            pltpu.VMEM((SUB, R), jnp.int32),
            pltpu.VMEM((SUB, R), jnp.int32),
            pltpu.VMEM((SUB, R), jnp.float32),
            pltpu.SemaphoreType.DMA,
        ],
    )
    return s2(outp, tcp, meta)


def kernel(pred, target, k):
    tgt = target.astype(jnp.int32)
    kthr = jnp.minimum(jnp.asarray(k, jnp.int32), 3)
    meta = jnp.zeros((SUB, R), jnp.int32)
    meta = meta.at[0].set(tgt)
    meta = meta.at[1].set(jnp.broadcast_to(kthr, (R,)))
    out = _run(pred.T, meta, tgt)
    return out[0, 0] / jnp.float32(target.shape[0])


# speedup vs baseline: 1.4883x; 1.4883x over previous
import jax
import jax.numpy as jnp
from jax import lax
from jax.experimental import pallas as pl
from jax.experimental.pallas import tpu as pltpu
from jax.experimental.pallas import tpu_sc as plsc

R = 128
N = 100000
L = 16
NC = 2
NS = 16
NW = NC * NS
NB = R // L
SUB = 8

SC_TILES = N // SUB
W_TILES = SC_TILES // NW
REM = SC_TILES - W_TILES * NW
SC_TILE0 = 0
CHUNK_TILES = 30
NCHUNKS = W_TILES // CHUNK_TILES
CHUNK_ROWS = CHUNK_TILES * SUB
NEG_TINY_BITS = -2147483647


def _sc_stage1(pt, tgt, outp, buf0, buf1, gath, tgtv, ebuf, outv,
               sem0, sem1, semg):
    wid = lax.axis_index("s") * NC + lax.axis_index("c")
    lanes = lax.iota(jnp.int32, L) + wid * jnp.int32(0)

    pltpu.sync_copy(tgt, tgtv)
    pltpu.async_copy(pt.at[tgtv], gath, semg).wait()

    t_vec, thr_lo, thr_hi, acc = [], [], [], []
    for m in range(NB):
        v = jnp.broadcast_to(jnp.float32(0.0), (L,)) + lanes.astype(
            jnp.float32) * jnp.float32(0.0)
        for l in range(L):
            row = gath[m * L + l, pl.ds(m * L, L)]
            v = jnp.where(lanes == jnp.int32(l), row, v)
        b = lax.bitcast_convert_type(v, jnp.int32)
        blo = jnp.where(v > jnp.float32(0.0), b - 1, b + 1)
        blo = jnp.where(v == jnp.float32(0.0), jnp.int32(NEG_TINY_BITS), blo)
        thr_lo.append(lax.bitcast_convert_type(blo, jnp.float32))
        thr_hi.append(v)
        t_vec.append(tgtv[pl.ds(m * L, L)])
        acc.append(lanes * jnp.int32(0))

    def chunk_src(c):
        row0 = (SC_TILE0 + wid * W_TILES + c * CHUNK_TILES) * SUB
        return pt.at[pl.ds(row0, CHUNK_ROWS), pl.ds(0, R)]

    bufs = (buf0, buf1)
    sems = (sem0, sem1)
    copies = {0: pltpu.async_copy(chunk_src(0), bufs[0], sems[0])}

    def count_block(buf, nrows, col0, accs, lo, hi):
        cs0 = jnp.broadcast_to(col0, (L,))

        @plsc.parallel_loop(0, nrows, 1, carry=tuple(accs) + (cs0,))
        def _loop(s, carry):
            a = list(carry[:NB])
            cs = carry[NB]
            for m in range(NB):
                x = buf[s, pl.ds(m * L, L)]
                thr = jnp.where(cs < t_vec[m], lo[m], hi[m])
                a[m] = a[m] + jnp.where(x > thr, jnp.int32(1), jnp.int32(0))
            return tuple(a) + (cs + jnp.int32(1),)

        return list(_loop[:NB])

    for c in range(NCHUNKS):
        p = c % 2
        if c + 1 < NCHUNKS:
            copies[c + 1] = pltpu.async_copy(chunk_src(c + 1),
                                             bufs[1 - p], sems[1 - p])
        copies[c].wait()
        acc = count_block(bufs[p], CHUNK_ROWS,
                          (SC_TILE0 + wid * W_TILES + c * CHUNK_TILES) * SUB,
                          acc, thr_lo, thr_hi)

    en = jnp.broadcast_to(wid < REM, (L,))
    row0 = (SC_TILE0 + NW * W_TILES + jnp.minimum(wid, REM - 1)) * SUB
    pltpu.async_copy(pt.at[pl.ds(row0, SUB), pl.ds(0, R)],
                     ebuf, semg).wait()
    inf = jnp.float32(float("inf"))
    lo_e = [jnp.where(en, thr_lo[m], inf) for m in range(NB)]
    hi_e = [jnp.where(en, thr_hi[m], inf) for m in range(NB)]
    acc = count_block(ebuf, SUB, row0, acc, lo_e, hi_e)

    for m in range(NB):
        outv[0, pl.ds(m * L, L)] = acc[m]
    pltpu.sync_copy(outv, outp.at[pl.ds(0, SUB),
                                  pl.ds(pl.multiple_of(wid * R, R), R)])


def _tc_stage2(pv_ref, kv_ref, o_ref):
    rank = pv_ref[0:1, 0:R].astype(jnp.int32)
    for w in range(1, NW):
        rank = rank + pv_ref[0:1, w * R:(w + 1) * R]
    hit = (rank < kv_ref[0:1, 0:R]).astype(jnp.float32)
    total = jnp.sum(hit)
    o_ref[...] = jnp.broadcast_to(total, (SUB, R))


@jax.jit
def _run(pt, tgt, kv):
    mesh = plsc.VectorSubcoreMesh(core_axis_name="c", subcore_axis_name="s")
    params = pltpu.CompilerParams(needs_layout_passes=False,
                                  use_tc_tiling_on_sc=True)
    s1 = pl.kernel(
        _sc_stage1,
        out_type=jax.ShapeDtypeStruct((SUB, NW * R), jnp.int32),
        mesh=mesh,
        compiler_params=params,
        scratch_types=[
            pltpu.VMEM((CHUNK_ROWS, R), jnp.float32),
            pltpu.VMEM((CHUNK_ROWS, R), jnp.float32),
            pltpu.VMEM((R, R), jnp.float32),
            pltpu.VMEM((R,), jnp.int32),
            pltpu.VMEM((SUB, R), jnp.float32),
            pltpu.VMEM((SUB, R), jnp.int32),
            pltpu.SemaphoreType.DMA,
            pltpu.SemaphoreType.DMA,
            pltpu.SemaphoreType.DMA,
        ],
    )
    outp = s1(pt, tgt)

    return pl.pallas_call(
        _tc_stage2,
        out_shape=jax.ShapeDtypeStruct((SUB, R), jnp.float32),
    )(outp, kv)


def kernel(pred, target, k):
    tgt = target.astype(jnp.int32)
    kthr = jnp.minimum(jnp.asarray(k, jnp.int32), 3)
    kv = jnp.broadcast_to(kthr, (SUB, R))
    out = _run(pred.T, tgt, kv)
    return out[0, 0] / jnp.float32(target.shape[0])


# speedup vs baseline: 1.5820x; 1.0629x over previous
import jax
import jax.numpy as jnp
from jax import lax
from jax.experimental import pallas as pl
from jax.experimental.pallas import tpu as pltpu
from jax.experimental.pallas import tpu_sc as plsc

R = 128
N = 100000
L = 16
NC = 2
NS = 16
NW = NC * NS
NB = R // L
SUB = 8

SC_TILES = N // SUB
W_TILES = SC_TILES // NW
REM = SC_TILES - W_TILES * NW
SC_TILE0 = 0
CHUNK_TILES = 30
NCHUNKS = W_TILES // CHUNK_TILES
CHUNK_ROWS = CHUNK_TILES * SUB
NEG_TINY_BITS = -2147483647


def _sc_stage1(pt, tgt, kv, outp, buf0, buf1, gath, tgtv, ebuf, outv,
               sem0, sem1, semg):
    wid = lax.axis_index("s") * NC + lax.axis_index("c")
    lanes = lax.iota(jnp.int32, L) + wid * jnp.int32(0)

    pltpu.sync_copy(tgt, tgtv)
    pltpu.async_copy(pt.at[tgtv], gath, semg).wait()

    t_vec, thr_lo, thr_hi, acc = [], [], [], []
    for m in range(NB):
        v = jnp.broadcast_to(jnp.float32(0.0), (L,)) + lanes.astype(
            jnp.float32) * jnp.float32(0.0)
        for l in range(L):
            row = gath[m * L + l, pl.ds(m * L, L)]
            v = jnp.where(lanes == jnp.int32(l), row, v)
        b = lax.bitcast_convert_type(v, jnp.int32)
        blo = jnp.where(v > jnp.float32(0.0), b - 1, b + 1)
        blo = jnp.where(v == jnp.float32(0.0), jnp.int32(NEG_TINY_BITS), blo)
        thr_lo.append(lax.bitcast_convert_type(blo, jnp.float32))
        thr_hi.append(v)
        t_vec.append(tgtv[pl.ds(m * L, L)])
        acc.append(lanes * jnp.int32(0))

    def chunk_src(c):
        row0 = (SC_TILE0 + wid * W_TILES + c * CHUNK_TILES) * SUB
        return pt.at[pl.ds(row0, CHUNK_ROWS), pl.ds(0, R)]

    bufs = (buf0, buf1)
    sems = (sem0, sem1)
    copies = {0: pltpu.async_copy(chunk_src(0), bufs[0], sems[0])}
    erow0 = (SC_TILE0 + NW * W_TILES + jnp.minimum(wid, REM - 1)) * SUB
    ecopy = pltpu.async_copy(pt.at[pl.ds(erow0, SUB), pl.ds(0, R)],
                             ebuf, semg)

    def count_block(buf, nrows, col0, accs, lo, hi):
        cs0 = jnp.broadcast_to(col0, (L,))

        @plsc.parallel_loop(0, nrows, 1, carry=tuple(accs) + (cs0,))
        def _loop(s, carry):
            a = list(carry[:NB])
            cs = carry[NB]
            for m in range(NB):
                x = buf[s, pl.ds(m * L, L)]
                thr = jnp.where(cs < t_vec[m], lo[m], hi[m])
                a[m] = a[m] + jnp.where(x > thr, jnp.int32(1), jnp.int32(0))
            return tuple(a) + (cs + jnp.int32(1),)

        return list(_loop[:NB])

    for c in range(NCHUNKS):
        p = c % 2
        if c + 1 < NCHUNKS:
            copies[c + 1] = pltpu.async_copy(chunk_src(c + 1),
                                             bufs[1 - p], sems[1 - p])
        copies[c].wait()
        acc = count_block(bufs[p], CHUNK_ROWS,
                          (SC_TILE0 + wid * W_TILES + c * CHUNK_TILES) * SUB,
                          acc, thr_lo, thr_hi)

    en = jnp.broadcast_to(wid < REM, (L,))
    ecopy.wait()
    inf = jnp.float32(float("inf"))
    lo_e = [jnp.where(en, thr_lo[m], inf) for m in range(NB)]
    hi_e = [jnp.where(en, thr_hi[m], inf) for m in range(NB)]
    acc = count_block(ebuf, SUB, erow0, acc, lo_e, hi_e)

    for m in range(NB):
        outv[0, pl.ds(m * L, L)] = acc[m]
    pltpu.sync_copy(outv, outp.at[pl.ds(0, SUB),
                                  pl.ds(pl.multiple_of(wid * R, R), R)])


def _tc_stage2(pv_ref, kv_ref, o_ref):
    rank = pv_ref[0:1, 0:R].astype(jnp.int32)
    for w in range(1, NW):
        rank = rank + pv_ref[0:1, w * R:(w + 1) * R]
    hit = (rank < kv_ref[0:1, 0:R]).astype(jnp.float32)
    total = jnp.sum(hit) * jnp.float32(1.0 / R)
    o_ref[...] = jnp.broadcast_to(total, (1, 1))


@jax.jit
def _run(pt, tgt, kv):
    mesh = plsc.VectorSubcoreMesh(core_axis_name="c", subcore_axis_name="s")
    params = pltpu.CompilerParams(needs_layout_passes=False,
                                  use_tc_tiling_on_sc=True)
    s1 = pl.kernel(
        _sc_stage1,
        out_type=jax.ShapeDtypeStruct((SUB, NW * R), jnp.int32),
        mesh=mesh,
        compiler_params=params,
        scratch_types=[
            pltpu.VMEM((CHUNK_ROWS, R), jnp.float32),
            pltpu.VMEM((CHUNK_ROWS, R), jnp.float32),
            pltpu.VMEM((R, R), jnp.float32),
            pltpu.VMEM((R,), jnp.int32),
            pltpu.VMEM((SUB, R), jnp.float32),
            pltpu.VMEM((SUB, R), jnp.int32),
            pltpu.SemaphoreType.DMA,
            pltpu.SemaphoreType.DMA,
            pltpu.SemaphoreType.DMA,
        ],
    )
    outp = s1(pt, tgt, kv)

    return pl.pallas_call(
        _tc_stage2,
        out_shape=jax.ShapeDtypeStruct((1, 1), jnp.float32),
    )(outp, kv)


def kernel(pred, target, k):
    tgt = target.astype(jnp.int32)
    kthr = jnp.minimum(jnp.asarray(k, jnp.int32), 3)
    kv = jnp.broadcast_to(kthr, (SUB, R))
    out = _run(pred.T, tgt, kv)
    return out[0, 0]
